# Initial kernel scaffold; baseline (speedup 1.0000x reference)
import jax
import jax.numpy as jnp
from jax.experimental import pallas as pl


def kernel(moving_img, target_img, edge_input, hex0, pseudo0, edge_index1, pseudo1, c1_g, c1_mu, c1_sigma, c1_root, c1_b, c1s_g, c1s_mu, c1s_sigma, c1s_root, c1s_b, c2_g, c2_mu, c2_sigma, c2_root, c2_b, c2s_g, c2s_mu, c2s_sigma, c2s_root, c2s_b):
    raise NotImplementedError("write your pallas kernel here")


# speedup vs baseline: 7.3656x; 7.3656x over previous
import functools

import jax
import jax.numpy as jnp
from jax import lax
from jax.experimental import pallas as pl
from jax.experimental.pallas import tpu as pltpu
from jax.experimental.pallas import tpu_sc as plsc

KK = 3
NC, NS = 2, 16
CHUNK = 128
ZR = 64

N0, E0 = 40962, 245760
N1, E1 = 10242, 61440
NP0 = 41984
NP1 = 11264
NH = 10368


def _sc_mesh():
    return plsc.VectorSubcoreMesh(core_axis_name="c", subcore_axis_name="s")


_SC_PARAMS = pltpu.CompilerParams(
    needs_layout_passes=False, use_tc_tiling_on_sc=False)


def _zero_acc(zbuf, acc, s, W, rpt):
    zv = jnp.zeros((16,), jnp.float32)

    def zrow(i, _):
        for j in range(W // 16):
            zbuf[i, pl.ds(j * 16, 16)] = zv
        return 0

    lax.fori_loop(0, ZR, zrow, 0)
    base = s * rpt

    def zblk(i, _):
        pltpu.sync_copy(zbuf, acc.at[pl.ds(base + i * ZR, ZR)])
        return 0

    lax.fori_loop(0, rpt // ZR, zblk, 0)


def _build_push(Np, in_c, E, small):
    W = 16 if small else 3 * in_c + 16
    rpt = Np // NS
    cpt = E // (NC * NS * CHUNK)

    @functools.partial(
        pl.kernel,
        out_type=jax.ShapeDtypeStruct((NC, Np, W), jnp.float32),
        mesh=_sc_mesh(),
        compiler_params=_SC_PARAMS,
        scratch_types=[
            pltpu.VMEM((CHUNK,), jnp.int32),
            pltpu.VMEM((CHUNK,), jnp.int32),
            pltpu.VMEM((CHUNK, KK), jnp.float32),
            pltpu.VMEM((CHUNK, in_c), jnp.float32),
            pltpu.VMEM((CHUNK, W), jnp.float32),
            pltpu.VMEM((ZR, W), jnp.float32),
            pltpu.VMEM_SHARED((Np, W), jnp.float32),
            pltpu.SemaphoreType.DMA,
        ],
    )
    def push(x_hbm, src_hbm, dst_hbm, w_hbm, out_hbm,
             sidx, didx, wbuf, xrows, obuf, zbuf, acc, sem):
        c = lax.axis_index("c")
        s = lax.axis_index("s")
        _zero_acc(zbuf, acc, s, W, rpt)
        plsc.subcore_barrier()

        iot = lax.iota(jnp.int32, 16)
        one_v = jnp.full((16,), 1.0, jnp.float32)
        if small:
            lane_c = jnp.remainder(iot, in_c)
            lane_k = jnp.minimum(iot // in_c, 2)
            pad_m = iot >= 3 * in_c

        t0 = (c * NS + s) * cpt

        def echunk(i, _):
            eb = (t0 + i) * CHUNK
            pltpu.sync_copy(src_hbm.at[pl.ds(eb, CHUNK)], sidx)
            pltpu.sync_copy(dst_hbm.at[pl.ds(eb, CHUNK)], didx)
            pltpu.sync_copy(w_hbm.at[pl.ds(eb, CHUNK)], wbuf)
            pltpu.async_copy(x_hbm.at[sidx], xrows, sem).wait()

            if small:
                def edge(e, _):
                    ev = jnp.full((16,), e, jnp.int32)
                    xv = plsc.load_gather(xrows, [ev, lane_c])
                    wv = plsc.load_gather(wbuf, [ev, lane_k])
                    obuf[e, :] = jnp.where(pad_m, one_v, wv * xv)
                    return 0
            else:
                def edge(e, _):
                    ev = jnp.full((16,), e, jnp.int32)
                    xvs = [xrows[e, pl.ds(j * 16, 16)] for j in range(in_c // 16)]
                    for k in range(KK):
                        wkv = plsc.load_gather(
                            wbuf, [ev, jnp.full((16,), k, jnp.int32)])
                        for j, xv in enumerate(xvs):
                            obuf[e, pl.ds(k * in_c + j * 16, 16)] = wkv * xv
                    obuf[e, pl.ds(3 * in_c, 16)] = one_v
                    return 0

            lax.fori_loop(0, CHUNK, edge, 0)
            pltpu.sync_copy(obuf, acc.at[didx], add=True)
            return 0

        lax.fori_loop(0, cpt, echunk, 0)
        plsc.subcore_barrier()
        base = s * rpt
        pltpu.sync_copy(acc.at[pl.ds(base, rpt)],
                        out_hbm.at[c, pl.ds(base, rpt)])

    return push


def _build_pull(Np, M, E):
    rpt = Np // NS
    cpt = E // (NC * NS * CHUNK)

    @functools.partial(
        pl.kernel,
        out_type=jax.ShapeDtypeStruct((NC, Np, M), jnp.float32),
        mesh=_sc_mesh(),
        compiler_params=_SC_PARAMS,
        scratch_types=[
            pltpu.VMEM((CHUNK,), jnp.int32),
            pltpu.VMEM((CHUNK,), jnp.int32),
            pltpu.VMEM((CHUNK, KK), jnp.float32),
            pltpu.VMEM((CHUNK, 3 * M), jnp.float32),
            pltpu.VMEM((CHUNK, M), jnp.float32),
            pltpu.VMEM((ZR, M), jnp.float32),
            pltpu.VMEM_SHARED((Np, M), jnp.float32),
            pltpu.SemaphoreType.DMA,
        ],
    )
    def pull(xg_hbm, src_hbm, dst_hbm, w_hbm, out_hbm,
             sidx, didx, wbuf, grows, obuf, zbuf, acc, sem):
        c = lax.axis_index("c")
        s = lax.axis_index("s")
        _zero_acc(zbuf, acc, s, M, rpt)
        plsc.subcore_barrier()

        t0 = (c * NS + s) * cpt

        def echunk(i, _):
            eb = (t0 + i) * CHUNK
            pltpu.sync_copy(src_hbm.at[pl.ds(eb, CHUNK)], sidx)
            pltpu.sync_copy(dst_hbm.at[pl.ds(eb, CHUNK)], didx)
            pltpu.sync_copy(w_hbm.at[pl.ds(eb, CHUNK)], wbuf)
            pltpu.async_copy(xg_hbm.at[sidx], grows, sem).wait()

            def edge(e, _):
                ev = jnp.full((16,), e, jnp.int32)
                wv = [plsc.load_gather(wbuf, [ev, jnp.full((16,), k, jnp.int32)])
                      for k in range(KK)]
                for j in range(M // 16):
                    mv = wv[0] * grows[e, pl.ds(j * 16, 16)]
                    mv = mv + wv[1] * grows[e, pl.ds(M + j * 16, 16)]
                    mv = mv + wv[2] * grows[e, pl.ds(2 * M + j * 16, 16)]
                    obuf[e, pl.ds(j * 16, 16)] = mv
                return 0

            lax.fori_loop(0, CHUNK, edge, 0)
            pltpu.sync_copy(obuf, acc.at[didx], add=True)
            return 0

        lax.fori_loop(0, cpt, echunk, 0)
        plsc.subcore_barrier()
        base = s * rpt
        pltpu.sync_copy(acc.at[pl.ds(base, rpt)],
                        out_hbm.at[c, pl.ds(base, rpt)])

    return pull


def _build_hex():
    NCH = NH // CHUNK

    @functools.partial(
        pl.kernel,
        out_type=jax.ShapeDtypeStruct((NH, 32), jnp.float32),
        mesh=_sc_mesh(),
        compiler_params=_SC_PARAMS,
        scratch_types=[
            pltpu.VMEM((CHUNK * 7,), jnp.int32),
            pltpu.VMEM((CHUNK * 7, 32), jnp.float32),
            pltpu.VMEM((CHUNK, 32), jnp.float32),
            pltpu.SemaphoreType.DMA,
        ],
    )
    def hexk(xl2_hbm, hexflat_hbm, out_hbm, hidx, hrows, obuf, sem):
        c = lax.axis_index("c")
        s = lax.axis_index("s")
        wid = c * NS + s
        iot = lax.iota(jnp.int32, 16)
        rows_c, cols_c = {}, {}
        for j in range(2):
            for t in range(7):
                fv = 7 * iot + (112 * j + t)
                rows_c[(j, t)] = fv // 32
                cols_c[(j, t)] = jnp.remainder(fv, 32)

        for ci in range((NCH + NC * NS - 1) // (NC * NS)):
            cid = wid + (NC * NS) * ci

            @pl.when(cid < NCH)
            def _():
                nb = cid * CHUNK
                pltpu.sync_copy(hexflat_hbm.at[pl.ds(nb * 7, CHUNK * 7)], hidx)
                pltpu.async_copy(xl2_hbm.at[hidx], hrows, sem).wait()

                def node(n, _):
                    b7 = n * 7
                    for j in range(2):
                        sv = None
                        for t in range(7):
                            g = plsc.load_gather(
                                hrows, [rows_c[(j, t)] + b7, cols_c[(j, t)]])
                            sv = g if sv is None else sv + g
                        obuf[n, pl.ds(j * 16, 16)] = sv * (1.0 / 7.0)
                    return 0

                lax.fori_loop(0, CHUNK, node, 0)
                pltpu.sync_copy(obuf, out_hbm.at[pl.ds(nb, CHUNK)])

    return hexk


def _gauss_tc(pseudo, mu, sigma):
    E = pseudo.shape[0]
    BLK = 1024

    def body(p_ref, mu_ref, sg_ref, o_ref):
        p = p_ref[...]
        mu = mu_ref[...]
        sg = sg_ref[...]
        cols = []
        for k in range(KK):
            t = jnp.zeros((BLK, 1), jnp.float32)
            for d in range(2):
                diff = p[:, d:d + 1] - mu[k:k + 1, d:d + 1]
                t = t + diff * diff / (1e-14 + sg[k:k + 1, d:d + 1] ** 2)
            cols.append(jnp.exp(-0.5 * t))
        o_ref[...] = jnp.concatenate(cols, axis=1)

    return pl.pallas_call(
        body,
        grid=(E // BLK,),
        in_specs=[pl.BlockSpec((BLK, 2), lambda i: (i, 0)),
                  pl.BlockSpec((KK, 2), lambda i: (0, 0)),
                  pl.BlockSpec((KK, 2), lambda i: (0, 0))],
        out_specs=pl.BlockSpec((BLK, KK), lambda i: (i, 0)),
        out_shape=jax.ShapeDtypeStruct((E, KK), jnp.float32),
    )(pseudo, mu, sigma)


def _lrelu(v):
    return jnp.where(v > 0, v, 0.2 * v)


def _post_prep_tc(Y, x, Gp, root, b, gs, n_rows, cnt_col):
    BLK = 512
    W = Y.shape[2]
    Cin = x.shape[1]
    M = root.shape[1]
    MS = gs.shape[1]
    grid = ((n_rows + BLK - 1) // BLK,)

    def body(y_ref, x_ref, G_ref, r_ref, b_ref, gs_ref, xl_ref, xg_ref, inv_ref):
        Ys = y_ref[0] + y_ref[1]
        cnt = Ys[:, cnt_col:cnt_col + 1]
        inv = 1.0 / jnp.maximum(cnt, 1.0)
        agg = jnp.dot(Ys, G_ref[...], preferred_element_type=jnp.float32)
        o = agg * inv + jnp.dot(x_ref[...], r_ref[...],
                                preferred_element_type=jnp.float32) + b_ref[...]
        o = _lrelu(o)
        xl_ref[...] = o
        xg_ref[...] = jnp.dot(o, gs_ref[...], preferred_element_type=jnp.float32)
        inv_ref[...] = inv

    return pl.pallas_call(
        body,
        grid=grid,
        in_specs=[pl.BlockSpec((NC, BLK, W), lambda i: (0, i, 0)),
                  pl.BlockSpec((BLK, Cin), lambda i: (i, 0)),
                  pl.BlockSpec((W, M), lambda i: (0, 0)),
                  pl.BlockSpec((Cin, M), lambda i: (0, 0)),
                  pl.BlockSpec((1, M), lambda i: (0, 0)),
                  pl.BlockSpec((M, MS), lambda i: (0, 0))],
        out_specs=[pl.BlockSpec((BLK, M), lambda i: (i, 0)),
                   pl.BlockSpec((BLK, MS), lambda i: (i, 0)),
                   pl.BlockSpec((BLK, 1), lambda i: (i, 0))],
        out_shape=[jax.ShapeDtypeStruct((n_rows, M), jnp.float32),
                   jax.ShapeDtypeStruct((n_rows, MS), jnp.float32),
                   jax.ShapeDtypeStruct((n_rows, 1), jnp.float32)],
    )(Y, x, Gp, root, b, gs)


def _post_tc(A, inv, x, root, b, n_rows):
    BLK = 512
    M = root.shape[1]
    Cin = x.shape[1]
    grid = ((n_rows + BLK - 1) // BLK,)

    def body(a_ref, inv_ref, x_ref, r_ref, b_ref, o_ref):
        agg = (a_ref[0] + a_ref[1]) * inv_ref[...]
        o = agg + jnp.dot(x_ref[...], r_ref[...],
                          preferred_element_type=jnp.float32) + b_ref[...]
        o_ref[...] = _lrelu(o)

    return pl.pallas_call(
        body,
        grid=grid,
        in_specs=[pl.BlockSpec((NC, BLK, M), lambda i: (0, i, 0)),
                  pl.BlockSpec((BLK, 1), lambda i: (i, 0)),
                  pl.BlockSpec((BLK, Cin), lambda i: (i, 0)),
                  pl.BlockSpec((Cin, M), lambda i: (0, 0)),
                  pl.BlockSpec((1, M), lambda i: (0, 0))],
        out_specs=pl.BlockSpec((BLK, M), lambda i: (i, 0)),
        out_shape=jax.ShapeDtypeStruct((n_rows, M), jnp.float32),
    )(A, inv, x, root, b)


def _gstack(g, in_c, M, W):
    Gs = g.reshape(in_c, KK, M).transpose(1, 0, 2).reshape(KK * in_c, M)
    return jnp.concatenate(
        [Gs, jnp.zeros((W - KK * in_c, M), jnp.float32)], axis=0)


def kernel(moving_img, target_img, edge_input, hex0, pseudo0, edge_index1,
           pseudo1, c1_g, c1_mu, c1_sigma, c1_root, c1_b, c1s_g, c1s_mu,
           c1s_sigma, c1s_root, c1s_b, c2_g, c2_mu, c2_sigma, c2_root, c2_b,
           c2s_g, c2s_mu, c2s_sigma, c2s_root, c2s_b):
    x_in = jnp.concatenate([moving_img, target_img], axis=1)
    src0, dst0 = edge_input[0], edge_input[1]
    src1, dst1 = edge_index1[0], edge_index1[1]
    hexflat = hex0.reshape(-1)

    w1 = _gauss_tc(pseudo0, c1_mu, c1_sigma)
    w1s = _gauss_tc(pseudo0, c1s_mu, c1s_sigma)
    w2 = _gauss_tc(pseudo1, c2_mu, c2_sigma)
    w2s = _gauss_tc(pseudo1, c2s_mu, c2s_sigma)

    Y0 = _build_push(NP0, 4, E0, small=True)(x_in, src0, dst0, w1)
    xl1, xg1s, inv0 = _post_prep_tc(
        Y0, x_in, _gstack(c1_g, 4, 32, 16), c1_root,
        c1_b.reshape(1, -1), c1s_g, N0, cnt_col=12)

    A1s = _build_pull(NP0, 32, E0)(xg1s, src0, dst0, w1s)
    xl2 = _post_tc(A1s, inv0, xl1, c1s_root, c1s_b.reshape(1, -1), N0)

    x1 = _build_hex()(xl2, hexflat)

    Y1 = _build_push(NP1, 32, E1, small=False)(x1, src1, dst1, w2)
    xl3, xg2s, inv1 = _post_prep_tc(
        Y1, x1, _gstack(c2_g, 32, 64, 112), c2_root,
        c2_b.reshape(1, -1), c2s_g, N1, cnt_col=96)

    A2s = _build_pull(NP1, 64, E1)(xg2s, src1, dst1, w2s)
    return _post_tc(A2s, inv1, xl3, c2s_root, c2s_b.reshape(1, -1), N1)
